# baseline (speedup 1.0000x reference)
import functools

import jax
import jax.numpy as jnp
import numpy as np
from jax import lax
from jax.experimental import pallas as pl
from jax.experimental.pallas import tpu as pltpu
from jax.experimental.pallas import tpu_sc as plsc

B = 128
V = 100000
SEGW = 512
NSEG = 196
MPAD = 256
RB = 32
CB = 12800
SPC = CB // SEGW
NSEL = 12
CPS = 5
NCH = NSEL * CPS
CW = 128
NTR = B * V // CW
K = 8
NW = 32
TEMP = np.float32(0.7)
TINY = np.float32(np.finfo(np.float32).tiny)
NEG_INF = np.float32(-np.inf)


def _k1_body(x_ref, ch_ref, base_ref, m_ref):
    gj = pl.program_id(0)
    x = x_ref[...]
    sfull = (V - (V // CB) * CB) // SEGW
    m_l = []
    for s in range(SPC):
        sl = x[s * SEGW:(s + 1) * SEGW, :]
        if s >= sfull:
            vrow = (lax.broadcasted_iota(jnp.int32, (SEGW, B), 0)
                    + gj * CB + s * SEGW)
            sl = jnp.where(vrow < V, sl, NEG_INF)
        m_l.append(jnp.max(sl, axis=0, keepdims=True))
    m_l.append(jnp.full((32 - SPC, B), NEG_INF, jnp.float32))
    m_ref[pl.ds(gj, 1), :, :] = jnp.concatenate(m_l, axis=0)[None]

    @pl.when(gj == (pl.num_programs(0) - 1))
    def _():
        iota = lax.broadcasted_iota(jnp.int32, (MPAD, B), 0)
        work = m_ref[...].reshape(MPAD, B)
        segs = []
        for _ in range(NSEL):
            m = jnp.max(work, axis=0, keepdims=True)
            sel = jnp.min(jnp.where(work == m, iota, MPAD), axis=0,
                          keepdims=True)
            work = jnp.where(iota == sel, NEG_INF, work)
            segs.append(sel)
        slot16 = jnp.concatenate(segs, axis=0)
        seg16 = (slot16 >> 5) * SPC + (slot16 & 31)
        seg = seg16.T

        sidx = lax.broadcasted_iota(jnp.int32, (B, NCH), 1) // CPS
        iota16 = lax.broadcasted_iota(jnp.int32, (B, NSEL), 1)
        seg_slot = jnp.zeros((B, NCH), jnp.int32)
        for t in range(NSEL):
            seg_t = jnp.sum(jnp.where(iota16 == t, seg, 0), axis=1,
                            keepdims=True)
            seg_slot = jnp.where(sidx == t, seg_t, seg_slot)
        row = lax.broadcasted_iota(jnp.int32, (B, NCH), 0)
        base = row * V + seg_slot * SEGW
        j = lax.broadcasted_iota(jnp.int32, (B, NCH), 1) % CPS
        ch_ref[...] = jnp.minimum((base >> 7) + j, NTR - 1)
        base_ref[...] = base


def _k1(logits_t):
    ncb = (V + CB - 1) // CB
    return pl.pallas_call(
        _k1_body,
        grid=(ncb,),
        in_specs=[pl.BlockSpec((CB, B), lambda j: (j, 0))],
        out_specs=(
            pl.BlockSpec((B, NCH), lambda j: (0, 0)),
            pl.BlockSpec((B, NCH), lambda j: (0, 0)),
        ),
        out_shape=(
            jax.ShapeDtypeStruct((B, NCH), jnp.int32),
            jax.ShapeDtypeStruct((B, NCH), jnp.int32),
        ),
        scratch_shapes=[pltpu.VMEM(((V + CB - 1) // CB, 32, B), jnp.float32)],
    )(logits_t)


_CH_PER_W = B * NCH // NW
_IDX_SPLIT = 128


@functools.cache
def _k2_gather():
    @functools.partial(
        pl.kernel,
        out_type=jax.ShapeDtypeStruct((B * NCH, CW), jnp.float32),
        mesh=plsc.VectorSubcoreMesh(core_axis_name="c", subcore_axis_name="s"),
        scratch_types=[
            pltpu.VMEM((_CH_PER_W,), jnp.int32),
            pltpu.VMEM((_CH_PER_W, CW), jnp.float32),
            pltpu.SemaphoreType.DMA,
        ],
    )
    def gather(table_hbm, idx_hbm, out_hbm, idx_v, rows_v, sem):
        wid = lax.axis_index("s") * 2 + lax.axis_index("c")
        base = wid * _CH_PER_W
        pltpu.sync_copy(idx_hbm.at[pl.ds(base, _CH_PER_W)], idx_v)
        copies = []
        off = 0
        while off < _CH_PER_W:
            n = min(_IDX_SPLIT, _CH_PER_W - off)
            copies.append(pltpu.async_copy(
                table_hbm.at[idx_v.at[pl.ds(off, n)]],
                rows_v.at[pl.ds(off, n)], sem))
            off += n
        for c in copies:
            c.wait()
        pltpu.sync_copy(rows_v, out_hbm.at[pl.ds(base, _CH_PER_W)])

    return gather


def _threefry_bits(flat_u32):
    rot0 = (13, 15, 26, 6)
    rot1 = (17, 29, 16, 24)
    ks0 = jnp.uint32(0)
    ks1 = jnp.uint32(42)
    ks2 = jnp.uint32(0 ^ 42 ^ 0x1BD11BDA)

    def rotl(v, d):
        return (v << jnp.uint32(d)) | (v >> jnp.uint32(32 - d))

    def rounds(x0, x1, rots):
        for r in rots:
            x0 = x0 + x1
            x1 = rotl(x1, r)
            x1 = x0 ^ x1
        return x0, x1

    x0 = jnp.zeros_like(flat_u32) + ks0
    x1 = flat_u32 + ks1
    x0, x1 = rounds(x0, x1, rot0)
    x0 = x0 + ks1
    x1 = x1 + ks2 + jnp.uint32(1)
    x0, x1 = rounds(x0, x1, rot1)
    x0 = x0 + ks2
    x1 = x1 + ks0 + jnp.uint32(2)
    x0, x1 = rounds(x0, x1, rot0)
    x0 = x0 + ks0
    x1 = x1 + ks1 + jnp.uint32(3)
    x0, x1 = rounds(x0, x1, rot1)
    x0 = x0 + ks1
    x1 = x1 + ks2 + jnp.uint32(4)
    x0, x1 = rounds(x0, x1, rot0)
    x0 = x0 + ks2
    x1 = x1 + ks0 + jnp.uint32(5)
    return x0 ^ x1


def _k3_body(g_ref, base_ref, oh_ref, tok_ref, cols_ref, vals_ref):
    g3 = g_ref[...]
    base = base_ref[...]
    rowv = lax.broadcasted_iota(jnp.int32, (B, 1), 0) * V
    base3 = base[:, :, None]
    ch3 = (base3 >> 7) + lax.broadcasted_iota(jnp.int32, (B, NCH, CW), 1) % CPS
    pos3 = ch3 * CW + lax.broadcasted_iota(jnp.int32, (B, NCH, CW), 2)
    d3 = pos3 - base3
    col3 = base3 - rowv[:, :, None] + d3
    valid = (d3 >= 0) & (d3 < SEGW) & (col3 < V) & (ch3 < NTR)
    work = jnp.where(valid, g3, NEG_INF).reshape(B, NCH * CW)

    iota_g = lax.broadcasted_iota(jnp.int32, (B, NCH * CW), 1)
    iota64 = lax.broadcasted_iota(jnp.int32, (B, NCH), 1)

    vals_l, cols_l = [], []
    for _ in range(K):
        m = jnp.max(work, axis=1, keepdims=True)
        gi = jnp.min(jnp.where(work == m, iota_g, NCH * CW), axis=1,
                     keepdims=True)
        work = jnp.where(iota_g == gi, NEG_INF, work)
        slot = gi >> 7
        off = gi - slot * CW
        base_s = jnp.sum(jnp.where(iota64 == slot, base, 0), axis=1,
                         keepdims=True)
        col = ((base_s >> 7) + slot % CPS) * CW + off - rowv
        vals_l.append(m)
        cols_l.append(col)
    vals8 = jnp.concatenate(vals_l, axis=1)
    cols8 = jnp.concatenate(cols_l, axis=1)

    scaled = vals8 / TEMP
    oh = oh_ref[...]
    kth = jnp.sum(scaled * oh, axis=1, keepdims=True)
    kept = scaled >= kth
    rowmax = scaled[:, 0:1]
    e = jnp.where(kept, jnp.exp(scaled - rowmax), np.float32(0.0))
    denom = jnp.sum(e, axis=1, keepdims=True)
    pvals = e / denom

    row8 = lax.broadcasted_iota(jnp.int32, (B, K), 0)
    flat = (row8 * V + cols8).astype(jnp.uint32)
    bits = _threefry_bits(flat)
    fb = (bits >> jnp.uint32(9)) | jnp.uint32(0x3F800000)
    floats = lax.bitcast_convert_type(fb, jnp.float32) - np.float32(1.0)
    u = jnp.maximum(TINY, floats + TINY)
    gum = -jnp.log(-jnp.log(u))

    score = jnp.where(kept, scaled + gum, NEG_INF)
    iota8 = lax.broadcasted_iota(jnp.int32, (B, K), 1)
    ms = jnp.max(score, axis=1, keepdims=True)
    slot = jnp.min(jnp.where(score == ms, iota8, K), axis=1, keepdims=True)
    tok_ref[...] = jnp.sum(jnp.where(iota8 == slot, cols8, 0), axis=1,
                           keepdims=True)
    cols_ref[...] = cols8
    vals_ref[...] = pvals


def _k3(gathered3, basem, onehot):
    return pl.pallas_call(
        _k3_body,
        out_shape=(
            jax.ShapeDtypeStruct((B, 1), jnp.int32),
            jax.ShapeDtypeStruct((B, K), jnp.int32),
            jax.ShapeDtypeStruct((B, K), jnp.float32),
        ),
    )(gathered3, basem, onehot)


_VROWS_W = V // NW
_VSUB = 320
_SUBS = [_VSUB] * (_VROWS_W // _VSUB) + (
    [_VROWS_W % _VSUB] if _VROWS_W % _VSUB else [])


@functools.cache
def _k4_scatter():
    @functools.partial(
        pl.kernel,
        out_type=jax.ShapeDtypeStruct((V * B,), jnp.float32),
        mesh=plsc.VectorSubcoreMesh(core_axis_name="c", subcore_axis_name="s"),
        scratch_types=[
            pltpu.VMEM((_VSUB * B,), jnp.float32),
            pltpu.VMEM((_VSUB * B,), jnp.float32),
            pltpu.VMEM((B * K,), jnp.int32),
            pltpu.VMEM((B * K,), jnp.float32),
            pltpu.SemaphoreType.DMA,
            pltpu.SemaphoreType.DMA,
        ],
        compiler_params=pltpu.CompilerParams(needs_layout_passes=False),
    )
    def scatter(cols_hbm, vals_hbm, out_hbm, zb0, zb1, cols_v, vals_v,
                sem0, sem1):
        wid = lax.axis_index("s") * 2 + lax.axis_index("c")
        pltpu.sync_copy(cols_hbm, cols_v)
        pltpu.sync_copy(vals_hbm, vals_v)
        lane_b = lax.iota(jnp.int32, 16) // K
        zbufs = (zb0, zb1)
        sems = (sem0, sem1)

        def _zero(zbuf):
            def body(i, carry):
                for u in range(8):
                    zbuf[pl.ds(i * 128 + u * 16, 16)] = \
                        jnp.zeros((16,), jnp.float32)
                return carry
            lax.fori_loop(0, _VSUB * B // 128, body, 0)

        _zero(zb0)
        _zero(zb1)

        def _scatter(zbuf, s, nrows, restoring):
            lo = (wid * _VROWS_W + s * _VSUB) * B

            def body(i, carry):
                for u in range(4):
                    iv = i * 4 + u
                    cv = cols_v[pl.ds(iv * 16, 16)]
                    vv = vals_v[pl.ds(iv * 16, 16)]
                    flat = cv * B + iv * 2 + lane_b
                    msk = (flat >= lo) & (flat < lo + nrows * B) \
                        & (vv > np.float32(0.0))
                    lidx = jnp.where(msk, flat - lo, 0)
                    put = (jnp.zeros((16,), jnp.float32) if restoring else vv)
                    plsc.store_scatter(zbuf, [lidx], put, mask=msk)
                return carry

            lax.fori_loop(0, B * K // 64, body, 0)

        copies = [None, None]
        for s, nrows in enumerate(_SUBS):
            bi = s % 2
            if copies[bi] is not None:
                copies[bi].wait()
                _scatter(zbufs[bi], s - 2, _SUBS[s - 2], True)
            _scatter(zbufs[bi], s, nrows, False)
            lo = (wid * _VROWS_W + s * _VSUB) * B
            copies[bi] = pltpu.async_copy(
                zbufs[bi].at[pl.ds(0, nrows * B)],
                out_hbm.at[pl.ds(lo, nrows * B)], sems[bi])
        for c in copies:
            if c is not None:
                c.wait()

    return scatter


def kernel(logits, top_k):
    ch, basem = _k1(logits.T)

    table = logits.reshape(NTR, CW)
    gathered = _k2_gather()(table, ch.reshape(B * NCH))

    onehot = jnp.broadcast_to(
        (jnp.arange(K, dtype=jnp.int32)[None, :]
         == jnp.asarray(top_k, jnp.int32) - 1).astype(jnp.float32), (B, K))
    tok, cols, vals = _k3(gathered.reshape(B, NCH, CW), basem, onehot)

    probs_t = _k4_scatter()(cols.reshape(B * K), vals.reshape(B * K))
    return tok[:, 0], probs_t.reshape(V, B).T


# speedup vs baseline: 1.0018x; 1.0018x over previous
import functools

import jax
import jax.numpy as jnp
import numpy as np
from jax import lax
from jax.experimental import pallas as pl
from jax.experimental.pallas import tpu as pltpu
from jax.experimental.pallas import tpu_sc as plsc

B = 128
V = 100000
SEGW = 512
NSEG = 196
MPAD = 256
RB = 32
CB = 12800
SPC = CB // SEGW
NSEL = 16
CPS = 5
NCH = NSEL * CPS
CW = 128
NTR = B * V // CW
K = 8
NW = 32
TEMP = np.float32(0.7)
TINY = np.float32(np.finfo(np.float32).tiny)
NEG_INF = np.float32(-np.inf)


def _k1_body(x_ref, ch_ref, base_ref, m_ref):
    gj = pl.program_id(0)
    x = x_ref[...]
    sfull = (V - (V // CB) * CB) // SEGW
    m_l = []
    for s in range(SPC):
        sl = x[s * SEGW:(s + 1) * SEGW, :]
        if s >= sfull:
            vrow = (lax.broadcasted_iota(jnp.int32, (SEGW, B), 0)
                    + gj * CB + s * SEGW)
            sl = jnp.where(vrow < V, sl, NEG_INF)
        m_l.append(jnp.max(sl, axis=0, keepdims=True))
    m_l.append(jnp.full((32 - SPC, B), NEG_INF, jnp.float32))
    m_ref[pl.ds(gj, 1), :, :] = jnp.concatenate(m_l, axis=0)[None]

    @pl.when(gj == (pl.num_programs(0) - 1))
    def _():
        iota = lax.broadcasted_iota(jnp.int32, (MPAD, B), 0)
        work = m_ref[...].reshape(MPAD, B)
        segs = []
        for _ in range(NSEL):
            m = jnp.max(work, axis=0, keepdims=True)
            sel = jnp.min(jnp.where(work == m, iota, MPAD), axis=0,
                          keepdims=True)
            work = jnp.where(iota == sel, NEG_INF, work)
            segs.append(sel)
        slot16 = jnp.concatenate(segs, axis=0)
        seg16 = (slot16 >> 5) * SPC + (slot16 & 31)
        seg = seg16.T

        sidx = lax.broadcasted_iota(jnp.int32, (B, NCH), 1) // CPS
        iota16 = lax.broadcasted_iota(jnp.int32, (B, NSEL), 1)
        seg_slot = jnp.zeros((B, NCH), jnp.int32)
        for t in range(NSEL):
            seg_t = jnp.sum(jnp.where(iota16 == t, seg, 0), axis=1,
                            keepdims=True)
            seg_slot = jnp.where(sidx == t, seg_t, seg_slot)
        row = lax.broadcasted_iota(jnp.int32, (B, NCH), 0)
        base = row * V + seg_slot * SEGW
        j = lax.broadcasted_iota(jnp.int32, (B, NCH), 1) % CPS
        ch_ref[...] = jnp.minimum((base >> 7) + j, NTR - 1)
        base_ref[...] = base


def _k1(logits_t):
    ncb = (V + CB - 1) // CB
    return pl.pallas_call(
        _k1_body,
        grid=(ncb,),
        in_specs=[pl.BlockSpec((CB, B), lambda j: (j, 0))],
        out_specs=(
            pl.BlockSpec((B, NCH), lambda j: (0, 0)),
            pl.BlockSpec((B, NCH), lambda j: (0, 0)),
        ),
        out_shape=(
            jax.ShapeDtypeStruct((B, NCH), jnp.int32),
            jax.ShapeDtypeStruct((B, NCH), jnp.int32),
        ),
        scratch_shapes=[pltpu.VMEM(((V + CB - 1) // CB, 32, B), jnp.float32)],
    )(logits_t)


_CH_PER_W = B * NCH // NW
_IDX_SPLIT = 128


@functools.cache
def _k2_gather():
    @functools.partial(
        pl.kernel,
        out_type=jax.ShapeDtypeStruct((B * NCH, CW), jnp.float32),
        mesh=plsc.VectorSubcoreMesh(core_axis_name="c", subcore_axis_name="s"),
        scratch_types=[
            pltpu.VMEM((_CH_PER_W,), jnp.int32),
            pltpu.VMEM((_CH_PER_W, CW), jnp.float32),
            pltpu.SemaphoreType.DMA,
        ],
    )
    def gather(table_hbm, idx_hbm, out_hbm, idx_v, rows_v, sem):
        wid = lax.axis_index("s") * 2 + lax.axis_index("c")
        base = wid * _CH_PER_W
        pltpu.sync_copy(idx_hbm.at[pl.ds(base, _CH_PER_W)], idx_v)
        copies = []
        off = 0
        while off < _CH_PER_W:
            n = min(_IDX_SPLIT, _CH_PER_W - off)
            copies.append(pltpu.async_copy(
                table_hbm.at[idx_v.at[pl.ds(off, n)]],
                rows_v.at[pl.ds(off, n)], sem))
            off += n
        for c in copies:
            c.wait()
        pltpu.sync_copy(rows_v, out_hbm.at[pl.ds(base, _CH_PER_W)])

    return gather


def _threefry_bits(flat_u32):
    rot0 = (13, 15, 26, 6)
    rot1 = (17, 29, 16, 24)
    ks0 = jnp.uint32(0)
    ks1 = jnp.uint32(42)
    ks2 = jnp.uint32(0 ^ 42 ^ 0x1BD11BDA)

    def rotl(v, d):
        return (v << jnp.uint32(d)) | (v >> jnp.uint32(32 - d))

    def rounds(x0, x1, rots):
        for r in rots:
            x0 = x0 + x1
            x1 = rotl(x1, r)
            x1 = x0 ^ x1
        return x0, x1

    x0 = jnp.zeros_like(flat_u32) + ks0
    x1 = flat_u32 + ks1
    x0, x1 = rounds(x0, x1, rot0)
    x0 = x0 + ks1
    x1 = x1 + ks2 + jnp.uint32(1)
    x0, x1 = rounds(x0, x1, rot1)
    x0 = x0 + ks2
    x1 = x1 + ks0 + jnp.uint32(2)
    x0, x1 = rounds(x0, x1, rot0)
    x0 = x0 + ks0
    x1 = x1 + ks1 + jnp.uint32(3)
    x0, x1 = rounds(x0, x1, rot1)
    x0 = x0 + ks1
    x1 = x1 + ks2 + jnp.uint32(4)
    x0, x1 = rounds(x0, x1, rot0)
    x0 = x0 + ks2
    x1 = x1 + ks0 + jnp.uint32(5)
    return x0 ^ x1


def _k3_body(g_ref, base_ref, oh_ref, tok_ref, cols_ref, vals_ref):
    g3 = g_ref[...]
    base = base_ref[...]
    rowv = lax.broadcasted_iota(jnp.int32, (B, 1), 0) * V
    base3 = base[:, :, None]
    ch3 = (base3 >> 7) + lax.broadcasted_iota(jnp.int32, (B, NCH, CW), 1) % CPS
    pos3 = ch3 * CW + lax.broadcasted_iota(jnp.int32, (B, NCH, CW), 2)
    d3 = pos3 - base3
    col3 = base3 - rowv[:, :, None] + d3
    valid = (d3 >= 0) & (d3 < SEGW) & (col3 < V) & (ch3 < NTR)
    work = jnp.where(valid, g3, NEG_INF).reshape(B, NCH * CW)

    iota_g = lax.broadcasted_iota(jnp.int32, (B, NCH * CW), 1)
    iota64 = lax.broadcasted_iota(jnp.int32, (B, NCH), 1)

    vals_l, cols_l = [], []
    for _ in range(K):
        m = jnp.max(work, axis=1, keepdims=True)
        gi = jnp.min(jnp.where(work == m, iota_g, NCH * CW), axis=1,
                     keepdims=True)
        work = jnp.where(iota_g == gi, NEG_INF, work)
        slot = gi >> 7
        off = gi - slot * CW
        base_s = jnp.sum(jnp.where(iota64 == slot, base, 0), axis=1,
                         keepdims=True)
        col = ((base_s >> 7) + slot % CPS) * CW + off - rowv
        vals_l.append(m)
        cols_l.append(col)
    vals8 = jnp.concatenate(vals_l, axis=1)
    cols8 = jnp.concatenate(cols_l, axis=1)

    scaled = vals8 / TEMP
    oh = oh_ref[...]
    kth = jnp.sum(scaled * oh, axis=1, keepdims=True)
    kept = scaled >= kth
    rowmax = scaled[:, 0:1]
    e = jnp.where(kept, jnp.exp(scaled - rowmax), np.float32(0.0))
    denom = jnp.sum(e, axis=1, keepdims=True)
    pvals = e / denom

    row8 = lax.broadcasted_iota(jnp.int32, (B, K), 0)
    flat = (row8 * V + cols8).astype(jnp.uint32)
    bits = _threefry_bits(flat)
    fb = (bits >> jnp.uint32(9)) | jnp.uint32(0x3F800000)
    floats = lax.bitcast_convert_type(fb, jnp.float32) - np.float32(1.0)
    u = jnp.maximum(TINY, floats + TINY)
    gum = -jnp.log(-jnp.log(u))

    score = jnp.where(kept, scaled + gum, NEG_INF)
    iota8 = lax.broadcasted_iota(jnp.int32, (B, K), 1)
    ms = jnp.max(score, axis=1, keepdims=True)
    slot = jnp.min(jnp.where(score == ms, iota8, K), axis=1, keepdims=True)
    tok_ref[...] = jnp.sum(jnp.where(iota8 == slot, cols8, 0), axis=1,
                           keepdims=True)
    cols_ref[...] = cols8
    vals_ref[...] = pvals


def _k3(gathered3, basem, onehot):
    return pl.pallas_call(
        _k3_body,
        out_shape=(
            jax.ShapeDtypeStruct((B, 1), jnp.int32),
            jax.ShapeDtypeStruct((B, K), jnp.int32),
            jax.ShapeDtypeStruct((B, K), jnp.float32),
        ),
    )(gathered3, basem, onehot)


_VROWS_W = V // NW
_VSUB = 320
_SUBS = [_VSUB] * (_VROWS_W // _VSUB) + (
    [_VROWS_W % _VSUB] if _VROWS_W % _VSUB else [])


@functools.cache
def _k4_scatter():
    @functools.partial(
        pl.kernel,
        out_type=jax.ShapeDtypeStruct((V * B,), jnp.float32),
        mesh=plsc.VectorSubcoreMesh(core_axis_name="c", subcore_axis_name="s"),
        scratch_types=[
            pltpu.VMEM((_VSUB * B,), jnp.float32),
            pltpu.VMEM((_VSUB * B,), jnp.float32),
            pltpu.VMEM((B * K,), jnp.int32),
            pltpu.VMEM((B * K,), jnp.float32),
            pltpu.SemaphoreType.DMA,
            pltpu.SemaphoreType.DMA,
        ],
        compiler_params=pltpu.CompilerParams(needs_layout_passes=False),
    )
    def scatter(cols_hbm, vals_hbm, out_hbm, zb0, zb1, cols_v, vals_v,
                sem0, sem1):
        wid = lax.axis_index("s") * 2 + lax.axis_index("c")
        pltpu.sync_copy(cols_hbm, cols_v)
        pltpu.sync_copy(vals_hbm, vals_v)
        lane_b = lax.iota(jnp.int32, 16) // K
        zbufs = (zb0, zb1)
        sems = (sem0, sem1)

        def _zero(zbuf):
            def body(i, carry):
                for u in range(8):
                    zbuf[pl.ds(i * 128 + u * 16, 16)] = \
                        jnp.zeros((16,), jnp.float32)
                return carry
            lax.fori_loop(0, _VSUB * B // 128, body, 0)

        _zero(zb0)
        _zero(zb1)

        def _scatter(zbuf, s, nrows, restoring):
            lo = (wid * _VROWS_W + s * _VSUB) * B

            def body(i, carry):
                for u in range(4):
                    iv = i * 4 + u
                    cv = cols_v[pl.ds(iv * 16, 16)]
                    vv = vals_v[pl.ds(iv * 16, 16)]
                    flat = cv * B + iv * 2 + lane_b
                    msk = (flat >= lo) & (flat < lo + nrows * B) \
                        & (vv > np.float32(0.0))
                    lidx = jnp.where(msk, flat - lo, 0)
                    put = (jnp.zeros((16,), jnp.float32) if restoring else vv)
                    plsc.store_scatter(zbuf, [lidx], put, mask=msk)
                return carry

            lax.fori_loop(0, B * K // 64, body, 0)

        copies = [None, None]
        for s, nrows in enumerate(_SUBS):
            bi = s % 2
            if copies[bi] is not None:
                copies[bi].wait()
                _scatter(zbufs[bi], s - 2, _SUBS[s - 2], True)
            _scatter(zbufs[bi], s, nrows, False)
            lo = (wid * _VROWS_W + s * _VSUB) * B
            copies[bi] = pltpu.async_copy(
                zbufs[bi].at[pl.ds(0, nrows * B)],
                out_hbm.at[pl.ds(lo, nrows * B)], sems[bi])
        for c in copies:
            if c is not None:
                c.wait()

    return scatter


def kernel(logits, top_k):
    ch, basem = _k1(logits.T)

    table = logits.reshape(NTR, CW)
    gathered = _k2_gather()(table, ch.reshape(B * NCH))

    onehot = jnp.broadcast_to(
        (jnp.arange(K, dtype=jnp.int32)[None, :]
         == jnp.asarray(top_k, jnp.int32) - 1).astype(jnp.float32), (B, K))
    tok, cols, vals = _k3(gathered.reshape(B, NCH, CW), basem, onehot)

    probs_t = _k4_scatter()(cols.reshape(B * K), vals.reshape(B * K))
    return tok[:, 0], probs_t.reshape(V, B).T
